# baseline (speedup 1.0000x reference)
import functools

import jax
import jax.numpy as jnp
from jax import lax
from jax.experimental import pallas as pl
from jax.experimental.pallas import tpu as pltpu
from jax.experimental.pallas import tpu_sc as plsc

N_VOX = 160000
N_RAYS = 10000
D = 256
NC = 2
NS = 16
NW = NC * NS
RPW = 320
R_PAD = NW * RPW
CH = 64
NVEC = D // 16
NEG_INF = float("-inf")
ZERO = 0.0


def _segmax_sc(feat, idx, bounds):
    mesh = plsc.VectorSubcoreMesh(
        core_axis_name="c", subcore_axis_name="s", num_cores=NC, num_subcores=NS
    )

    @functools.partial(
        pl.kernel,
        out_type=jax.ShapeDtypeStruct((R_PAD * D,), jnp.float32),
        mesh=mesh,
        compiler_params=pltpu.CompilerParams(use_tc_tiling_on_sc=True),
        scratch_types=[
            pltpu.VMEM((RPW * D,), jnp.float32),
            pltpu.VMEM((2, CH, D), jnp.float32),
            pltpu.VMEM((2, CH), jnp.int32),
            pltpu.VMEM((NW + 16,), jnp.int32),
            pltpu.SemaphoreType.DMA,
            pltpu.SemaphoreType.DMA,
            pltpu.SemaphoreType.DMA,
            pltpu.SemaphoreType.DMA,
        ],
    )
    def k(feat_hbm, idx_hbm, bounds_hbm, out_hbm, acc, fbuf, ibuf, bbuf,
          fsem0, fsem1, isem0, isem1):
        wid = lax.axis_index("s") * NC + lax.axis_index("c")
        ray_lo = wid * RPW
        pltpu.sync_copy(bounds_hbm, bbuf)
        bvec = bbuf[pl.ds(wid, 16)]
        row_start = bvec[0]
        row_end = bvec[1]

        zeros = jnp.zeros((16,), jnp.float32)

        def init_row(i, _):
            for c in range(NVEC):
                acc[pl.ds(i * D + c * 16, 16)] = zeros
            return 0

        lax.fori_loop(0, RPW, init_row, 0)

        c_lo = row_start // CH
        c_hi = (row_end + CH - 1) // CH

        def start_dma(kk):
            r0 = kk * CH

            @pl.when(kk % 2 == 0)
            def _():
                pltpu.async_copy(
                    idx_hbm.at[pl.ds(r0, CH)], ibuf.at[0], isem0)
                pltpu.async_copy(
                    feat_hbm.at[pl.ds(r0, CH), :], fbuf.at[0], fsem0)

            @pl.when(kk % 2 == 1)
            def _():
                pltpu.async_copy(
                    idx_hbm.at[pl.ds(r0, CH)], ibuf.at[1], isem1)
                pltpu.async_copy(
                    feat_hbm.at[pl.ds(r0, CH), :], fbuf.at[1], fsem1)

        def wait_dma(kk):
            @pl.when(kk % 2 == 0)
            def _():
                pltpu.make_async_copy(
                    idx_hbm.at[pl.ds(0, CH)], ibuf.at[0], isem0).wait()
                pltpu.make_async_copy(
                    feat_hbm.at[pl.ds(0, CH), :], fbuf.at[0], fsem0).wait()

            @pl.when(kk % 2 == 1)
            def _():
                pltpu.make_async_copy(
                    idx_hbm.at[pl.ds(0, CH)], ibuf.at[1], isem1).wait()
                pltpu.make_async_copy(
                    feat_hbm.at[pl.ds(0, CH), :], fbuf.at[1], fsem1).wait()

        cf_lo = (row_start + CH - 1) // CH
        cf_hi = row_end // CH
        b_lo = jnp.minimum(jnp.maximum(cf_lo, c_lo), c_hi)
        b_hi = jnp.maximum(jnp.minimum(cf_hi, c_hi), b_lo)

        @pl.when(c_lo < c_hi)
        def _():
            start_dma(c_lo)

        def make_chunk_body(masked):
            def chunk_body(kk, carry):
                @pl.when(kk + 1 < c_hi)
                def _():
                    start_dma(kk + 1)

                wait_dma(kk)
                pp = kk % 2

                if masked:
                    def group_body(g, carry):
                        tvec = ibuf[pp, pl.ds(g * 16, 16)] - ray_lo
                        for l in range(16):
                            row = g * 16 + l
                            t = tvec[l]
                            cur = carry[0]
                            regs = carry[1:]
                            valid = (t >= 0) & (t < RPW)
                            boundary = valid & (t != cur)

                            @pl.when(boundary & (cur >= 0))
                            def _():
                                for c in range(NVEC):
                                    acc[pl.ds(cur * D + c * 16, 16)] = regs[c]

                            addf = jnp.where(valid, ZERO, NEG_INF)
                            addr = jnp.where(boundary, NEG_INF, ZERO)
                            new_regs = tuple(
                                jnp.maximum(
                                    fbuf[pp, row, pl.ds(c * 16, 16)] + addf,
                                    regs[c] + addr)
                                for c in range(NVEC)
                            )
                            new_cur = jnp.where(boundary, t, cur)
                            carry = (new_cur,) + new_regs
                        return carry

                    return lax.fori_loop(0, CH // 16, group_body, carry)

                def group_body(g, carry):
                    tvec = ibuf[pp, pl.ds(g * 16, 16)] - ray_lo
                    for l in range(16):
                        row = g * 16 + l
                        t = tvec[l]
                        cur = carry[0]
                        regs = carry[1:]
                        boundary = t != cur

                        @pl.when(boundary & (cur >= 0))
                        def _():
                            for c in range(NVEC):
                                acc[pl.ds(cur * D + c * 16, 16)] = regs[c]

                        addr = jnp.where(boundary, NEG_INF, ZERO)
                        new_regs = tuple(
                            jnp.maximum(
                                fbuf[pp, row, pl.ds(c * 16, 16)],
                                regs[c] + addr)
                            for c in range(NVEC)
                        )
                        new_cur = jnp.where(boundary, t, cur)
                        carry = (new_cur,) + new_regs
                    return carry

                return lax.fori_loop(0, CH // 16, group_body, carry)
            return chunk_body

        init = (jnp.int32(-1),) + tuple(
            jnp.full((16,), NEG_INF, jnp.float32) for _ in range(NVEC)
        )
        carry = lax.fori_loop(c_lo, b_lo, make_chunk_body(True), init)
        carry = lax.fori_loop(b_lo, b_hi, make_chunk_body(False), carry)
        final = lax.fori_loop(b_hi, c_hi, make_chunk_body(True), carry)
        cur = final[0]
        regs = final[1:]

        @pl.when(cur >= 0)
        def _():
            for c in range(NVEC):
                acc[pl.ds(cur * D + c * 16, 16)] = regs[c]

        pltpu.sync_copy(acc, out_hbm.at[pl.ds(ray_lo * D, RPW * D)])

    return k(feat, idx, bounds)


def _linear_tc(ray_feat, W, b2d):
    BM = 2000

    def mm(x_ref, w_ref, b_ref, o_ref):
        y = lax.dot_general(
            x_ref[...], w_ref[...], (((1,), (1,)), ((), ())),
            preferred_element_type=jnp.float32,
        )
        o_ref[...] = jnp.maximum(y + b_ref[...], 0.0)

    return pl.pallas_call(
        mm,
        grid=(N_RAYS // BM,),
        in_specs=[
            pl.BlockSpec((BM, D), lambda i: (i, 0)),
            pl.BlockSpec((D, D), lambda i: (0, 0)),
            pl.BlockSpec((1, D), lambda i: (0, 0)),
        ],
        out_specs=pl.BlockSpec((BM, D), lambda i: (i, 0)),
        out_shape=jax.ShapeDtypeStruct((N_RAYS, D), jnp.float32),
    )(ray_feat, W, b2d)


def kernel(inp_feat, vox2ray_idx, W, b):
    idx = vox2ray_idx.astype(jnp.int32)
    ray_starts = jnp.minimum(
        jnp.arange(NW, dtype=jnp.int32) * RPW, N_RAYS)
    bounds = jnp.sum(
        idx[:, None] < ray_starts[None, :], axis=0, dtype=jnp.int32)
    bounds = jnp.concatenate([bounds, jnp.full((16,), N_VOX, jnp.int32)])
    rf = _segmax_sc(inp_feat, idx, bounds)
    return _linear_tc(rf.reshape(R_PAD, D), W, b.reshape(1, D))
